# baseline (speedup 1.0000x reference)
import functools

import jax
import jax.numpy as jnp
from jax import lax
from jax.experimental import pallas as pl
from jax.experimental.pallas import tpu as pltpu
from jax.experimental.pallas import tpu_sc as plsc

N = 50000
E = 800000
NP = 51200
R = NP // 128
PADBASE = 50176
EP = 802816
B1 = EP // 32
NB1 = B1 // 128
B64 = EP // 16
NB64 = B64 // 128
ZR = NP // 16

_mesh = plsc.VectorSubcoreMesh(core_axis_name="c", subcore_axis_name="s")
_f32 = jnp.float32
_sc_params = pltpu.CompilerParams(
    use_tc_tiling_on_sc=False, needs_layout_passes=False)


K1 = 7
NG1 = NB1 // K1
K64 = 2
NG64 = NB64 // K64


def _sdeg_body(dst2_hbm, z_hbm, out_hbm, dstbuf, valbuf, acc, isem, ssem):
    c = lax.axis_index("c")
    s = lax.axis_index("s")
    ones16 = jnp.ones((16,), _f32)
    for b in range(K1):
        for v in range(8):
            valbuf[b, pl.ds(v * 16, 16)] = ones16
    pltpu.sync_copy(z_hbm, acc.at[pl.ds(s * ZR, ZR)])
    plsc.subcore_barrier()
    row_base = (c * 16 + s) * (B1 // 128)

    def fire_idx(o):
        pltpu.async_copy(dst2_hbm.at[pl.ds(row_base + o * K1, K1)],
                         dstbuf.at[o % 2], isem)

    def drain_idx(o):
        pltpu.make_async_copy(dst2_hbm.at[pl.ds(0, K1)],
                              dstbuf.at[o % 2], isem).wait()

    def fire_scatters(o):
        for b in range(K1):
            pltpu.async_copy(valbuf.at[b], acc.at[dstbuf.at[o % 2, b]],
                             ssem, add=True)

    def drain_scatters(o):
        for b in range(K1):
            pltpu.make_async_copy(valbuf.at[b], acc.at[dstbuf.at[o % 2, b]],
                                  ssem).wait()

    fire_idx(0)

    def step(o, carry):
        drain_idx(o)

        @pl.when(o > 0)
        def _():
            drain_scatters(o - 1)

        @pl.when(o + 1 < NG1)
        def _():
            fire_idx(o + 1)

        fire_scatters(o)
        return carry

    lax.fori_loop(0, NG1, step, 0)
    drain_scatters(NG1 - 1)
    plsc.subcore_barrier()
    pltpu.sync_copy(acc.at[pl.ds(s * ZR, ZR)], out_hbm.at[c].at[pl.ds(s * ZR, ZR)])


_sdeg = functools.partial(
    pl.kernel,
    mesh=_mesh,
    compiler_params=_sc_params,
    out_type=jax.ShapeDtypeStruct((2, NP), _f32),
    scratch_types=[
        pltpu.VMEM((2, K1, 128), jnp.int32),
        pltpu.VMEM((K1, 128), _f32),
        pltpu.VMEM_SHARED((NP,), _f32),
        pltpu.SemaphoreType.DMA,
        pltpu.SemaphoreType.DMA,
    ],
)(_sdeg_body)


def _s1_like_body(tables_hbm, src_hbm, dst2_hbm, z_hbm, outs_hbm,
                  tables_v, srcbuf, dstbuf, valbufs, accs, isem, ssem):
    c = lax.axis_index("c")
    s = lax.axis_index("s")
    for t_hbm, t_v in zip(tables_hbm, tables_v):
        pltpu.sync_copy(t_hbm, t_v)
    for acc in accs:
        pltpu.sync_copy(z_hbm, acc.at[pl.ds(s * ZR, ZR)])
    plsc.subcore_barrier()
    base = (c * 16 + s) * B1
    row_base = base // 128

    def fire_idx(o):
        par = o % 2
        pltpu.async_copy(src_hbm.at[pl.ds(base + o * (K1 * 128), K1 * 128)],
                         srcbuf.at[par], isem)
        pltpu.async_copy(dst2_hbm.at[pl.ds(row_base + o * K1, K1)],
                         dstbuf.at[par], isem)

    def drain_idx(o):
        par = o % 2
        pltpu.make_async_copy(src_hbm.at[pl.ds(0, K1 * 128)],
                              srcbuf.at[par], isem).wait()
        pltpu.make_async_copy(dst2_hbm.at[pl.ds(0, K1)],
                              dstbuf.at[par], isem).wait()

    def gather_group(par):
        for b in range(K1):
            for v in range(8):
                idx = srcbuf[par, pl.ds(b * 128 + v * 16, 16)]
                for table_v, valbuf in zip(tables_v, valbufs):
                    valbuf[par, b, pl.ds(v * 16, 16)] = plsc.load_gather(
                        table_v, [idx])

    def fire_scatters(o):
        par = o % 2
        for b in range(K1):
            for valbuf, acc in zip(valbufs, accs):
                pltpu.async_copy(valbuf.at[par, b],
                                 acc.at[dstbuf.at[par, b]], ssem, add=True)

    def drain_scatters(o):
        par = o % 2
        for b in range(K1):
            for valbuf, acc in zip(valbufs, accs):
                pltpu.make_async_copy(valbuf.at[par, b],
                                      acc.at[dstbuf.at[par, b]], ssem).wait()

    fire_idx(0)

    def step(o, carry):
        par = o % 2
        drain_idx(o)
        gather_group(par)

        @pl.when(o > 0)
        def _():
            drain_scatters(o - 1)

        @pl.when(o + 1 < NG1)
        def _():
            fire_idx(o + 1)

        fire_scatters(o)
        return carry

    lax.fori_loop(0, NG1, step, 0)
    drain_scatters(NG1 - 1)
    plsc.subcore_barrier()
    for out_hbm, acc in zip(outs_hbm, accs):
        pltpu.sync_copy(acc.at[pl.ds(s * ZR, ZR)],
                        out_hbm.at[c].at[pl.ds(s * ZR, ZR)])


def _s1_body(table_hbm, src_hbm, dst2_hbm, z_hbm, out_hbm,
             table_v, srcbuf, dstbuf, valbuf, acc, isem, ssem):
    _s1_like_body([table_hbm], src_hbm, dst2_hbm, z_hbm, [out_hbm],
                  [table_v], srcbuf, dstbuf, [valbuf], [acc], isem, ssem)


_s1 = functools.partial(
    pl.kernel,
    mesh=_mesh,
    compiler_params=_sc_params,
    out_type=jax.ShapeDtypeStruct((2, NP), _f32),
    scratch_types=[
        pltpu.VMEM((NP,), _f32),
        pltpu.VMEM((2, K1 * 128), jnp.int32),
        pltpu.VMEM((2, K1, 128), jnp.int32),
        pltpu.VMEM((2, K1, 128), _f32),
        pltpu.VMEM_SHARED((NP,), _f32),
        pltpu.SemaphoreType.DMA,
        pltpu.SemaphoreType.DMA,
    ],
)(_s1_body)


def _s64_body(u3_hbm, src_hbm, dst2_hbm, z_hbm, out_hbm,
              srcbuf, dstbuf, gbuf, acc, isem, gsem, ssem):
    c = lax.axis_index("c")
    s = lax.axis_index("s")
    pltpu.sync_copy(z_hbm, acc.at[pl.ds(s * ZR, ZR)])
    plsc.subcore_barrier()
    base = s * B64
    row_base = base // 128

    def fire_idx(o):
        i3 = o % 3
        pltpu.async_copy(src_hbm.at[pl.ds(base + o * (K64 * 128), K64 * 128)],
                         srcbuf.at[i3], isem)
        pltpu.async_copy(dst2_hbm.at[pl.ds(row_base + o * K64, K64)],
                         dstbuf.at[i3], isem)

    def drain_idx(o):
        i3 = o % 3
        pltpu.make_async_copy(src_hbm.at[pl.ds(0, K64 * 128)],
                              srcbuf.at[i3], isem).wait()
        pltpu.make_async_copy(dst2_hbm.at[pl.ds(0, K64)],
                              dstbuf.at[i3], isem).wait()

    def fire_gathers(o):
        i3 = o % 3
        i2 = o % 2
        for b in range(K64):
            pltpu.async_copy(
                u3_hbm.at[c].at[srcbuf.at[i3, pl.ds(b * 128, 128)]],
                gbuf.at[i2, b], gsem)

    def drain_gathers(o):
        i3 = o % 3
        i2 = o % 2
        for b in range(K64):
            pltpu.make_async_copy(
                u3_hbm.at[c].at[srcbuf.at[i3, pl.ds(b * 128, 128)]],
                gbuf.at[i2, b], gsem).wait()

    def fire_scatters(o):
        i3 = o % 3
        i2 = o % 2
        for b in range(K64):
            pltpu.async_copy(gbuf.at[i2, b], acc.at[dstbuf.at[i3, b]],
                             ssem, add=True)

    def drain_scatters(o):
        i3 = o % 3
        i2 = o % 2
        for b in range(K64):
            pltpu.make_async_copy(gbuf.at[i2, b], acc.at[dstbuf.at[i3, b]],
                                  ssem).wait()

    fire_idx(0)

    def step(o, carry):
        drain_idx(o)

        @pl.when(o >= 1)
        def _():
            drain_gathers(o - 1)
            fire_scatters(o - 1)

        @pl.when(o >= 2)
        def _():
            drain_scatters(o - 2)

        fire_gathers(o)

        @pl.when(o + 1 < NG64)
        def _():
            fire_idx(o + 1)

        return carry

    lax.fori_loop(0, NG64, step, 0)
    drain_gathers(NG64 - 1)
    fire_scatters(NG64 - 1)
    drain_scatters(NG64 - 2)
    drain_scatters(NG64 - 1)
    plsc.subcore_barrier()
    pltpu.sync_copy(acc.at[pl.ds(s * ZR, ZR)], out_hbm.at[c].at[pl.ds(s * ZR, ZR)])


_s64 = functools.partial(
    pl.kernel,
    mesh=_mesh,
    compiler_params=_sc_params,
    out_type=jax.ShapeDtypeStruct((2, NP, 32), _f32),
    scratch_types=[
        pltpu.VMEM((3, K64 * 128), jnp.int32),
        pltpu.VMEM((3, K64, 128), jnp.int32),
        pltpu.VMEM((2, K64, 128, 32), _f32),
        pltpu.VMEM_SHARED((NP, 32), _f32),
        pltpu.SemaphoreType.DMA,
        pltpu.SemaphoreType.DMA,
        pltpu.SemaphoreType.DMA,
    ],
)(_s64_body)


def _t1_body(degp_ref, xp_ref, dinv_ref, u0_ref):
    deg = degp_ref[0] + degp_ref[1] + 1.0
    dinv = lax.rsqrt(deg)
    dinv_ref[...] = dinv
    u0_ref[...] = dinv * xp_ref[...]


def _t1(degp, xp):
    return pl.pallas_call(
        _t1_body,
        out_shape=[jax.ShapeDtypeStruct((R, 128), _f32),
                   jax.ShapeDtypeStruct((R, 128), _f32)],
    )(degp, xp)


def _eye128():
    rid = lax.broadcasted_iota(jnp.int32, (128, 128), 0)
    cid = lax.broadcasted_iota(jnp.int32, (128, 128), 1)
    return rid == cid


def _col(eye, rowvec):
    return jnp.sum(jnp.where(eye, rowvec, 0.0), axis=1, keepdims=True)


RB = 8
TG = R // RB


def _t2_body(s0a_ref, s0b_ref, u0_ref, dinv_ref, w1_ref, b1_ref, w2_ref,
             u2a_ref, u2b_ref):
    w1 = w1_ref[...]
    eye = _eye128()
    h1_chunks = []
    dcol_chunks = []
    for k in range(RB):
        dinv = dinv_ref[k]
        p = dinv * (s0a_ref[k] + s0b_ref[k] + u0_ref[k])
        pcol = _col(eye, p)
        h1_chunks.append(jnp.maximum(pcol * w1 + b1_ref[...], 0.0))
        dcol_chunks.append(_col(eye, dinv))
    h1 = jnp.concatenate(h1_chunks, axis=0)
    dcol = jnp.concatenate(dcol_chunks, axis=0)
    t2 = jnp.dot(h1, w2_ref[...], preferred_element_type=_f32)
    u2 = dcol * t2
    u2a_ref[...] = jnp.stack([u2[:, 0:32], u2[:, 32:64]], axis=0)
    u2b_ref[...] = jnp.stack([u2[:, 64:96], u2[:, 96:128]], axis=0)


def _t2(s0a, s0b, u0, dinv, W1, b1, W2):
    full = lambda shape: pl.BlockSpec(shape, lambda r: (0,) * len(shape))
    row = pl.BlockSpec((RB, 1, 128), lambda r: (r, 0, 0))
    out = pl.BlockSpec((2, RB * 128, 32), lambda r: (0, r, 0))
    return pl.pallas_call(
        _t2_body,
        grid=(TG,),
        in_specs=[
            row, row, row, row,
            full((1, 256)),
            full((1, 256)),
            full((256, 128)),
        ],
        out_specs=[out, out],
        out_shape=[jax.ShapeDtypeStruct((2, NP, 32), _f32),
                   jax.ShapeDtypeStruct((2, NP, 32), _f32)],
    )(s0a, s0b, u0, dinv, W1, b1, W2)


def _t3_body(s2a_ref, s2b_ref, u2a_ref, u2b_ref, dinv_ref,
             w3_ref, b2_ref, u3_ref):
    s2f = jnp.concatenate(
        [s2a_ref[0], s2a_ref[1], s2b_ref[0], s2b_ref[1]], axis=1)
    u2f = jnp.concatenate(
        [u2a_ref[0], u2a_ref[1], u2b_ref[0], u2b_ref[1]], axis=1)
    eye = _eye128()
    dcol = jnp.concatenate([_col(eye, dinv_ref[k]) for k in range(RB)], axis=0)
    h2 = jnp.maximum(dcol * (s2f + u2f) + b2_ref[...], 0.0)
    t3 = jnp.dot(h2, w3_ref[...], preferred_element_type=_f32)
    u3 = dcol * t3
    u3_ref[...] = jnp.stack([u3[:, :32], u3[:, 32:]], axis=0)


def _t3(s2a, s2b, u2a, u2b, dinv, W3, b2):
    full = lambda shape: pl.BlockSpec(shape, lambda r: (0,) * len(shape))
    blk = pl.BlockSpec((2, RB * 128, 32), lambda r: (0, r, 0))
    return pl.pallas_call(
        _t3_body,
        grid=(TG,),
        in_specs=[
            blk, blk, blk, blk,
            pl.BlockSpec((RB, 1, 128), lambda r: (r, 0, 0)),
            full((128, 64)),
            full((1, 128)),
        ],
        out_specs=pl.BlockSpec((2, RB * 128, 32), lambda r: (0, r, 0)),
        out_shape=jax.ShapeDtypeStruct((2, NP, 32), _f32),
    )(s2a, s2b, u2a, u2b, dinv, W3, b2)


def _t4_body(s3_ref, u3_ref, dinv_ref, b3_ref, w4_ref, u4_ref):
    s3f = jnp.concatenate([s3_ref[0], s3_ref[1]], axis=1)
    u3f = jnp.concatenate([u3_ref[0], u3_ref[1]], axis=1)
    eye = _eye128()
    dcol_chunks = [_col(eye, dinv_ref[k]) for k in range(RB)]
    dcol = jnp.concatenate(dcol_chunks, axis=0)
    h3 = jnp.maximum(dcol * (s3f + u3f) + b3_ref[...], 0.0)
    t4 = jnp.dot(h3, w4_ref[...], preferred_element_type=_f32)
    u4_col = dcol * t4
    for k in range(RB):
        chunk = u4_col[k * 128:(k + 1) * 128]
        u4_ref[k] = jnp.sum(jnp.where(eye, chunk, 0.0), axis=0, keepdims=True)


def _t4(s3, u3, dinv, b3, W4):
    full = lambda shape: pl.BlockSpec(shape, lambda r: (0,) * len(shape))
    return pl.pallas_call(
        _t4_body,
        grid=(TG,),
        in_specs=[
            pl.BlockSpec((2, RB * 128, 32), lambda r: (0, r, 0)),
            pl.BlockSpec((2, RB * 128, 32), lambda r: (0, r, 0)),
            pl.BlockSpec((RB, 1, 128), lambda r: (r, 0, 0)),
            full((1, 64)),
            full((64, 1)),
        ],
        out_specs=pl.BlockSpec((RB, 1, 128), lambda r: (r, 0, 0)),
        out_shape=jax.ShapeDtypeStruct((R, 1, 128), _f32),
    )(s3, u3, dinv, b3, W4)


def _t5_body(s4p_ref, u4_ref, dinv_ref, b4_ref, o_ref):
    o_ref[...] = (dinv_ref[...] * (s4p_ref[0] + s4p_ref[1] + u4_ref[...])
                  + b4_ref[...])


def _t5(s4p, u4, dinv, b4):
    return pl.pallas_call(
        _t5_body,
        out_shape=jax.ShapeDtypeStruct((R, 128), _f32),
    )(s4p, u4, dinv, b4)


def kernel(x, edge_index, W1, b1, W2, b2, W3, b3, W4, b4):
    xp = jnp.pad(x[:, 0], (0, NP - N)).reshape(R, 128)
    pad_src = jnp.zeros((EP - E,), jnp.int32)
    pad_dst = PADBASE + (jnp.arange(EP - E, dtype=jnp.int32) % 128)
    srcp = jnp.concatenate([edge_index[0], pad_src])
    dstp = jnp.concatenate([edge_index[1], pad_dst]).reshape(EP // 128, 128)
    z1 = jnp.zeros((ZR,), _f32)
    z64 = jnp.zeros((ZR, 32), _f32)

    degp = _sdeg(dstp, z1)
    dinv, u0 = _t1(degp.reshape(2, R, 128), xp)
    s0p = _s1(u0.reshape(NP), srcp, dstp, z1).reshape(2, R, 1, 128)
    dinv3 = dinv.reshape(R, 1, 128)
    u2a, u2b = _t2(s0p[0], s0p[1], u0.reshape(R, 1, 128), dinv3,
                   W1.reshape(1, 256), b1.reshape(1, 256), W2)
    s2a = _s64(u2a, srcp, dstp, z64)
    s2b = _s64(u2b, srcp, dstp, z64)
    u3 = _t3(s2a, s2b, u2a, u2b, dinv3, W3, b2.reshape(1, 128))
    s3 = _s64(u3, srcp, dstp, z64)
    u4 = _t4(s3, u3, dinv3, b3.reshape(1, 64), W4).reshape(R, 128)
    s4p = _s1(u4.reshape(NP), srcp, dstp, z1)
    o = _t5(s4p.reshape(2, R, 128), u4, dinv, b4.reshape(1, 1))
    return o.reshape(NP)[:N].reshape(N, 1)
